# baseline (speedup 1.0000x reference)
import functools

import numpy as np
import jax
import jax.numpy as jnp
from jax import lax
from jax.experimental import pallas as pl
from jax.experimental.pallas import tpu as pltpu
from jax.experimental.pallas import tpu_sc as plsc

_P = 0.5
_B, _S, _D = 4, 4096, 1024
_NC, _NS = 2, 16
_NW = _NC * _NS
_ROWS = _B * _S
_RPW = _ROWS // _NW
_CHUNK = 32
_NCHUNK = _RPW // _CHUNK
_NBUF = 3


def _src_rows() -> np.ndarray:
    rng = np.random.default_rng(0)
    mask = rng.random(_S - 1) < _P
    idxs = np.arange(_S - 1)[mask]
    perm = np.arange(_S)
    perm[idxs] = idxs + 1
    perm[idxs + 1] = idxs
    rows = np.arange(_B)[:, None] * _S + perm[None, :]
    return rows.astype(np.int32).reshape(_NW, _NCHUNK, _CHUNK)


_IDX = _src_rows()

_mesh = plsc.VectorSubcoreMesh(core_axis_name="c", subcore_axis_name="s")


@functools.partial(
    pl.kernel,
    mesh=_mesh,
    out_type=jax.ShapeDtypeStruct((_ROWS, _D), jnp.float32),
    scratch_types=[pltpu.VMEM((_NCHUNK, _CHUNK), jnp.int32)]
    + [pltpu.VMEM((_CHUNK, _D), jnp.float32) for _ in range(_NBUF)]
    + [pltpu.SemaphoreType.DMA for _ in range(2 * _NBUF)],
)
def _gather_rows(x_hbm, idx_hbm, out_hbm, idx_v, *scr):
    bufs = scr[:_NBUF]
    gsem = scr[_NBUF:2 * _NBUF]
    wsem = scr[2 * _NBUF:]
    wid = lax.axis_index("s") * _NC + lax.axis_index("c")
    base = wid * _RPW

    pltpu.sync_copy(idx_hbm.at[wid], idx_v)

    def gather(ci):
        return pltpu.async_copy(
            x_hbm.at[idx_v.at[ci]], bufs[ci % _NBUF], gsem[ci % _NBUF])

    gh = [None] * _NBUF
    for ci in range(_NBUF):
        gh[ci] = gather(ci)
    for ci in range(_NCHUNK):
        s = ci % _NBUF
        gh[s].wait()
        w = pltpu.async_copy(
            bufs[s], out_hbm.at[pl.ds(base + ci * _CHUNK, _CHUNK)], wsem[s])
        w.wait()
        nx = ci + _NBUF
        if nx < _NCHUNK:
            gh[s] = gather(nx)


@jax.jit
def kernel(x):
    out = _gather_rows(x.reshape(_ROWS, _D), jnp.asarray(_IDX))
    return out.reshape(_B, _S, _D)


# speedup vs baseline: 1.0100x; 1.0100x over previous
import functools

import numpy as np
import jax
import jax.numpy as jnp
from jax import lax
from jax.experimental import pallas as pl
from jax.experimental.pallas import tpu as pltpu
from jax.experimental.pallas import tpu_sc as plsc

_P = 0.5
_B, _S, _D = 4, 4096, 1024
_NC, _NS = 2, 16
_NW = _NC * _NS
_ROWS = _B * _S
_RPW = _ROWS // _NW
_CHUNK = 56
_CS = [56] * 9 + [8]
_OFF = [56 * i for i in range(9)] + [504]
_NCHUNK = len(_CS)
_NBUF = 2


def _src_rows() -> np.ndarray:
    rng = np.random.default_rng(0)
    mask = rng.random(_S - 1) < _P
    idxs = np.arange(_S - 1)[mask]
    perm = np.arange(_S)
    perm[idxs] = idxs + 1
    perm[idxs + 1] = idxs
    rows = (np.arange(_B)[:, None] * _S + perm[None, :]).astype(np.int32)
    rows = rows.reshape(_NW, _RPW)
    tbl = np.zeros((_NW, _NCHUNK, _CHUNK), dtype=np.int32)
    for ci, (o, n) in enumerate(zip(_OFF, _CS)):
        tbl[:, ci, :n] = rows[:, o:o + n]
    return tbl


_IDX = _src_rows()

_mesh = plsc.VectorSubcoreMesh(core_axis_name="c", subcore_axis_name="s")


@functools.partial(
    pl.kernel,
    mesh=_mesh,
    out_type=jax.ShapeDtypeStruct((_ROWS, _D), jnp.float32),
    scratch_types=[pltpu.VMEM((_NCHUNK, _CHUNK), jnp.int32)]
    + [pltpu.VMEM((_CHUNK, _D), jnp.float32) for _ in range(_NBUF)]
    + [pltpu.SemaphoreType.DMA for _ in range(2 * _NBUF)],
)
def _gather_rows(x_hbm, idx_hbm, out_hbm, idx_v, *scr):
    bufs = scr[:_NBUF]
    gsem = scr[_NBUF:2 * _NBUF]
    wsem = scr[2 * _NBUF:]
    wid = lax.axis_index("s") * _NC + lax.axis_index("c")
    base = wid * _RPW

    pltpu.sync_copy(idx_hbm.at[wid], idx_v)

    def gather(ci):
        s = ci % _NBUF
        n = _CS[ci]
        idx = idx_v.at[ci] if n == _CHUNK else idx_v.at[ci, pl.ds(0, n)]
        dst = bufs[s] if n == _CHUNK else bufs[s].at[pl.ds(0, n)]
        return pltpu.async_copy(x_hbm.at[idx], dst, gsem[s])

    gh = [None] * _NBUF
    for ci in range(_NBUF):
        gh[ci] = gather(ci)
    for ci in range(_NCHUNK):
        s = ci % _NBUF
        n = _CS[ci]
        src_buf = bufs[s] if n == _CHUNK else bufs[s].at[pl.ds(0, n)]
        gh[s].wait()
        w = pltpu.async_copy(
            src_buf, out_hbm.at[pl.ds(base + _OFF[ci], n)], wsem[s])
        w.wait()
        nx = ci + _NBUF
        if nx < _NCHUNK:
            gh[s] = gather(nx)


@jax.jit
def kernel(x):
    out = _gather_rows(x.reshape(_ROWS, _D), jnp.asarray(_IDX))
    return out.reshape(_B, _S, _D)
